# baseline (speedup 1.0000x reference)
import functools

import jax
import jax.numpy as jnp
from jax import lax
from jax.experimental import pallas as pl
from jax.experimental.pallas import tpu as pltpu
from jax.experimental.pallas import tpu_sc as plsc

N_EMBEDDINGS = 1000000
D_EMBEDDING = 16

_NC = 2
_NS = 16
_NW = _NC * _NS

_B = 4096
_S = 200
_LANE = 128
_GS = 4
_NG = _S // _GS


def _quantize_body(x_ref, idx_ref):
    x = x_ref[...]
    m = jnp.max(x, axis=1, keepdims=True)
    xn = x / m * float(N_EMBEDDINGS - 1)
    xn = jnp.where(xn < 0, 0.0, xn)
    idx_ref[...] = xn.astype(jnp.int32).T


def _quantize_t(x):
    return pl.pallas_call(
        _quantize_body,
        out_shape=jax.ShapeDtypeStruct((_S, _B), jnp.int32),
    )(x)


_NTILE = 7813
_NROWS_PAD = _NTILE * 128


def _make_relinearize():
    mesh = plsc.VectorSubcoreMesh(core_axis_name="c", subcore_axis_name="s")
    batch = 4
    n_step = 61

    @functools.partial(
        pl.kernel,
        mesh=mesh,
        out_type=jax.ShapeDtypeStruct((_NROWS_PAD * 16,), jnp.float32),
        compiler_params=pltpu.CompilerParams(
            use_tc_tiling_on_sc=True, needs_layout_passes=False),
        scratch_types=[
            pltpu.VMEM((16, batch * 128 + 1), jnp.float32),
            pltpu.VMEM((16, batch * 128 + 1), jnp.float32),
            pltpu.VMEM((16, batch * 128 + 1), jnp.float32),
            pltpu.VMEM((16, batch * 128 + 1), jnp.float32),
            pltpu.VMEM((batch * 2048,), jnp.float32),
            pltpu.VMEM((batch * 2048,), jnp.float32),
            pltpu.VMEM((batch * 2048,), jnp.float32),
            pltpu.VMEM((batch * 2048,), jnp.float32),
            pltpu.VMEM((16, 128), jnp.float32),
            pltpu.VMEM((2048,), jnp.float32),
            pltpu.SemaphoreType.DMA,
            pltpu.SemaphoreType.DMA,
        ],
    )
    def relin(tt_hbm, tail_hbm, lin_hbm, v_a, v_b, v_c, v_d,
              o_a, o_b, o_c, o_d, v_t, o_t, isem, wsem):
        wid = lax.axis_index("s") * _NC + lax.axis_index("c")
        iota = lax.iota(jnp.int32, 16)
        base = 244 * wid + jnp.minimum(wid, 4)

        def fire_in(lane, vbuf, width):
            for dt in range(2):
                pltpu.async_copy(
                    tt_hbm.at[pl.ds(dt * 8, 8), pl.ds(lane, width)],
                    vbuf.at[pl.ds(dt * 8, 8), pl.ds(0, width)], isem)

        def drain_in(vbuf, width):
            for dt in range(2):
                pltpu.make_async_copy(
                    tt_hbm.at[pl.ds(0, 8), pl.ds(0, width)],
                    vbuf.at[pl.ds(dt * 8, 8), pl.ds(0, width)], isem).wait()

        def transpose(vbuf, obuf, width):
            @plsc.parallel_loop(0, width, 8, unroll=2)
            def _(r0):
                for k in range(8):
                    v = plsc.load_gather(
                        vbuf, [iota, jnp.zeros((16,), jnp.int32) + (r0 + k)])
                    obuf[pl.ds((r0 + k) * 16, 16)] = v

        def fire_out(lane, obuf, width):
            pltpu.async_copy(obuf.at[pl.ds(0, width * 16)],
                             lin_hbm.at[pl.ds(lane * 16, width * 16)], wsem)

        def drain_out(obuf, width):
            pltpu.make_async_copy(obuf.at[pl.ds(0, width * 16)],
                                  lin_hbm.at[pl.ds(0, width * 16)], wsem).wait()

        def lane_of(t):
            return (base + t * batch) * 128

        def process(t, vbuf, obuf, fire_ahead, first):
            drain_in(vbuf, batch * 128)
            if not first:
                drain_out(obuf, batch * 128)
            transpose(vbuf, obuf, batch * 128)
            fire_out(lane_of(t), obuf, batch * 128)
            if fire_ahead:
                fire_in(lane_of(t + 4), vbuf, batch * 128)

        vbufs = [v_a, v_b, v_c, v_d]
        obufs = [o_a, o_b, o_c, o_d]
        for t in range(4):
            fire_in(lane_of(t), vbufs[t], batch * 128)
        for t in range(4):
            process(t, vbufs[t], obufs[t], fire_ahead=True, first=True)

        def body(i, carry):
            for q in range(4):
                process(4 * i + q, vbufs[q], obufs[q],
                        fire_ahead=True, first=False)
            return carry

        lax.fori_loop(1, 14, body, 0)
        process(56, v_a, o_a, fire_ahead=False, first=False)
        fire_in(lane_of(60), v_a, batch * 128)
        process(57, v_b, o_b, fire_ahead=False, first=False)
        process(58, v_c, o_c, fire_ahead=False, first=False)
        process(59, v_d, o_d, fire_ahead=False, first=False)
        process(60, v_a, o_a, fire_ahead=False, first=False)
        drain_out(o_b, batch * 128)
        drain_out(o_c, batch * 128)
        drain_out(o_d, batch * 128)
        drain_out(o_a, batch * 128)

        @pl.when(wid < 4)
        def _tail_full():
            lane = (base + n_step * batch) * 128
            for dt in range(2):
                pltpu.async_copy(
                    tt_hbm.at[pl.ds(dt * 8, 8), pl.ds(lane, 128)],
                    v_t.at[pl.ds(dt * 8, 8)], isem)
            for dt in range(2):
                pltpu.make_async_copy(
                    tt_hbm.at[pl.ds(0, 8), pl.ds(0, 128)],
                    v_t.at[pl.ds(dt * 8, 8)], isem).wait()

            @plsc.parallel_loop(0, 128, 1, unroll=8)
            def _(r0):
                v = plsc.load_gather(v_t, [iota, jnp.zeros((16,), jnp.int32) + r0])
                o_t[pl.ds(r0 * 16, 16)] = v
            pltpu.async_copy(o_t, lin_hbm.at[pl.ds(lane * 16, 2048)], wsem)
            pltpu.make_async_copy(o_t, lin_hbm.at[pl.ds(0, 2048)], wsem).wait()

        @pl.when(wid == 31)
        def _tail_partial():
            pltpu.sync_copy(tail_hbm, o_t.at[pl.ds(0, 1024)])
            pltpu.sync_copy(o_t.at[pl.ds(0, 1024)],
                            lin_hbm.at[pl.ds((_NTILE - 1) * 2048, 1024)])

    return relin


def _make_gather():
    mesh = plsc.VectorSubcoreMesh(core_axis_name="c", subcore_axis_name="s")
    n_out = _S * D_EMBEDDING * _B

    @functools.partial(
        pl.kernel,
        mesh=mesh,
        out_type=jax.ShapeDtypeStruct((n_out,), jnp.float32),
        compiler_params=pltpu.CompilerParams(
            use_tc_tiling_on_sc=False, needs_layout_passes=False),
        scratch_types=[
            pltpu.VMEM((_S, _LANE), jnp.int32),
            pltpu.VMEM((_GS * _LANE, D_EMBEDDING), jnp.float32),
            pltpu.VMEM((_GS * _LANE, D_EMBEDDING), jnp.float32),
            pltpu.VMEM((_GS * 2048,), jnp.float32),
            pltpu.VMEM((_GS * 2048,), jnp.float32),
            pltpu.SemaphoreType.DMA,
            pltpu.SemaphoreType.DMA,
        ],
    )
    def gather(table_hbm, idxt_hbm, out_hbm, idxv, r_a, r_b, stg_a, stg_b,
               gsem, wsem):
        wid = lax.axis_index("s") * _NC + lax.axis_index("c")
        iota = lax.iota(jnp.int32, 16)

        pltpu.sync_copy(idxt_hbm.at[:, pl.ds(wid * _LANE, _LANE)], idxv)

        def fire_g(g, rbuf):
            for j in range(_GS):
                pltpu.async_copy(
                    table_hbm.at[idxv.at[g * _GS + j]],
                    rbuf.at[pl.ds(j * _LANE, _LANE)], gsem)

        def drain_g(rbuf):
            for j in range(_GS):
                pltpu.make_async_copy(
                    table_hbm.at[idxv.at[0]],
                    rbuf.at[pl.ds(j * _LANE, _LANE)], gsem).wait()

        rows = [iota + c * 16 for c in range(8)]

        def transpose(rbuf, stg):
            @plsc.parallel_loop(0, _GS * D_EMBEDDING, 1, unroll=2)
            def _(i):
                j = i >> 4
                d = i & 15
                col = jnp.zeros((16,), jnp.int32) + d
                base = j * 2048 + d * 128
                joff = j << 7
                for c in range(8):
                    v = plsc.load_gather(rbuf, [rows[c] + joff, col])
                    stg[pl.ds(base + c * 16, 16)] = v

        def fire_w(g, stg):
            for j in range(_GS):
                s = g * _GS + j
                for dt in range(2):
                    pltpu.async_copy(
                        stg.at[pl.ds(j * 2048 + dt * 1024, 1024)],
                        out_hbm.at[pl.ds(((2 * s + dt) * 32 + wid) * 1024, 1024)],
                        wsem)

        def drain_w(stg):
            for j in range(_GS):
                for dt in range(2):
                    pltpu.make_async_copy(
                        stg.at[pl.ds(j * 2048 + dt * 1024, 1024)],
                        out_hbm.at[pl.ds(0, 1024)], wsem).wait()

        def process(g, rbuf, stg, fire_ahead, first):
            drain_g(rbuf)
            if not first:
                drain_w(stg)
            transpose(rbuf, stg)
            fire_w(g, stg)
            if fire_ahead:
                fire_g(g + 2, rbuf)

        fire_g(0, r_a)
        fire_g(1, r_b)
        process(0, r_a, stg_a, fire_ahead=True, first=True)
        process(1, r_b, stg_b, fire_ahead=True, first=True)

        def body(i, carry):
            process(2 * i, r_a, stg_a, fire_ahead=True, first=False)
            process(2 * i + 1, r_b, stg_b, fire_ahead=True, first=False)
            return carry

        lax.fori_loop(1, _NG // 2 - 1, body, 0)
        process(_NG - 2, r_a, stg_a, fire_ahead=False, first=False)
        process(_NG - 1, r_b, stg_b, fire_ahead=False, first=False)
        drain_w(stg_a)
        drain_w(stg_b)

    return gather


def kernel(x, table):
    idx_t = _quantize_t(x)
    tail = table[(_NTILE - 1) * 128:, :].reshape(1024)
    lin = _make_relinearize()(table.T, tail).reshape(_NROWS_PAD, 16)
    flat = _make_gather()(lin, idx_t)
    return (flat.reshape(_S, 2, 32, 8, _LANE)
            .transpose(2, 4, 0, 1, 3)
            .reshape(_B, _S, D_EMBEDDING))


# speedup vs baseline: 1.0031x; 1.0031x over previous
import functools

import jax
import jax.numpy as jnp
from jax import lax
from jax.experimental import pallas as pl
from jax.experimental.pallas import tpu as pltpu
from jax.experimental.pallas import tpu_sc as plsc

N_EMBEDDINGS = 1000000
D_EMBEDDING = 16

_NC = 2
_NS = 16
_NW = _NC * _NS

_B = 4096
_S = 200
_LANE = 128
_GS = 4
_NG = _S // _GS


def _quantize_body(x_ref, idx_ref):
    x = x_ref[...]
    m = jnp.max(x, axis=1, keepdims=True)
    xn = x / m * float(N_EMBEDDINGS - 1)
    xn = jnp.where(xn < 0, 0.0, xn)
    idx_ref[...] = xn.astype(jnp.int32).T


def _quantize_t(x):
    return pl.pallas_call(
        _quantize_body,
        out_shape=jax.ShapeDtypeStruct((_S, _B), jnp.int32),
    )(x)


_NTILE = 7813
_NROWS_PAD = _NTILE * 128


def _make_relinearize():
    mesh = plsc.VectorSubcoreMesh(core_axis_name="c", subcore_axis_name="s")
    batch = 4
    n_step = 61

    @functools.partial(
        pl.kernel,
        mesh=mesh,
        out_type=jax.ShapeDtypeStruct((_NROWS_PAD * 16,), jnp.float32),
        compiler_params=pltpu.CompilerParams(
            use_tc_tiling_on_sc=True, needs_layout_passes=False),
        scratch_types=[
            pltpu.VMEM((16, batch * 128 + 1), jnp.float32),
            pltpu.VMEM((16, batch * 128 + 1), jnp.float32),
            pltpu.VMEM((16, batch * 128 + 1), jnp.float32),
            pltpu.VMEM((16, batch * 128 + 1), jnp.float32),
            pltpu.VMEM((batch * 2048,), jnp.float32),
            pltpu.VMEM((batch * 2048,), jnp.float32),
            pltpu.VMEM((batch * 2048,), jnp.float32),
            pltpu.VMEM((batch * 2048,), jnp.float32),
            pltpu.VMEM((16, 128), jnp.float32),
            pltpu.VMEM((2048,), jnp.float32),
            pltpu.SemaphoreType.DMA,
            pltpu.SemaphoreType.DMA,
        ],
    )
    def relin(tt_hbm, tail_hbm, lin_hbm, v_a, v_b, v_c, v_d,
              o_a, o_b, o_c, o_d, v_t, o_t, isem, wsem):
        wid = lax.axis_index("s") * _NC + lax.axis_index("c")
        iota = lax.iota(jnp.int32, 16)
        base = 244 * wid + jnp.minimum(wid, 4)

        def fire_in(lane, vbuf, width):
            for dt in range(2):
                pltpu.async_copy(
                    tt_hbm.at[pl.ds(dt * 8, 8), pl.ds(lane, width)],
                    vbuf.at[pl.ds(dt * 8, 8), pl.ds(0, width)], isem)

        def drain_in(vbuf, width):
            for dt in range(2):
                pltpu.make_async_copy(
                    tt_hbm.at[pl.ds(0, 8), pl.ds(0, width)],
                    vbuf.at[pl.ds(dt * 8, 8), pl.ds(0, width)], isem).wait()

        def transpose(vbuf, obuf, width):
            @plsc.parallel_loop(0, width, 8, unroll=4)
            def _(r0):
                for k in range(8):
                    v = plsc.load_gather(
                        vbuf, [iota, jnp.zeros((16,), jnp.int32) + (r0 + k)])
                    obuf[pl.ds((r0 + k) * 16, 16)] = v

        def fire_out(lane, obuf, width):
            pltpu.async_copy(obuf.at[pl.ds(0, width * 16)],
                             lin_hbm.at[pl.ds(lane * 16, width * 16)], wsem)

        def drain_out(obuf, width):
            pltpu.make_async_copy(obuf.at[pl.ds(0, width * 16)],
                                  lin_hbm.at[pl.ds(0, width * 16)], wsem).wait()

        def lane_of(t):
            return (base + t * batch) * 128

        def process(t, vbuf, obuf, fire_ahead, first):
            drain_in(vbuf, batch * 128)
            if not first:
                drain_out(obuf, batch * 128)
            transpose(vbuf, obuf, batch * 128)
            fire_out(lane_of(t), obuf, batch * 128)
            if fire_ahead:
                fire_in(lane_of(t + 4), vbuf, batch * 128)

        vbufs = [v_a, v_b, v_c, v_d]
        obufs = [o_a, o_b, o_c, o_d]
        for t in range(4):
            fire_in(lane_of(t), vbufs[t], batch * 128)
        for t in range(4):
            process(t, vbufs[t], obufs[t], fire_ahead=True, first=True)

        def body(i, carry):
            for q in range(4):
                process(4 * i + q, vbufs[q], obufs[q],
                        fire_ahead=True, first=False)
            return carry

        lax.fori_loop(1, 14, body, 0)
        process(56, v_a, o_a, fire_ahead=False, first=False)
        fire_in(lane_of(60), v_a, batch * 128)
        process(57, v_b, o_b, fire_ahead=False, first=False)
        process(58, v_c, o_c, fire_ahead=False, first=False)
        process(59, v_d, o_d, fire_ahead=False, first=False)
        process(60, v_a, o_a, fire_ahead=False, first=False)
        drain_out(o_b, batch * 128)
        drain_out(o_c, batch * 128)
        drain_out(o_d, batch * 128)
        drain_out(o_a, batch * 128)

        @pl.when(wid < 4)
        def _tail_full():
            lane = (base + n_step * batch) * 128
            for dt in range(2):
                pltpu.async_copy(
                    tt_hbm.at[pl.ds(dt * 8, 8), pl.ds(lane, 128)],
                    v_t.at[pl.ds(dt * 8, 8)], isem)
            for dt in range(2):
                pltpu.make_async_copy(
                    tt_hbm.at[pl.ds(0, 8), pl.ds(0, 128)],
                    v_t.at[pl.ds(dt * 8, 8)], isem).wait()

            @plsc.parallel_loop(0, 128, 1, unroll=8)
            def _(r0):
                v = plsc.load_gather(v_t, [iota, jnp.zeros((16,), jnp.int32) + r0])
                o_t[pl.ds(r0 * 16, 16)] = v
            pltpu.async_copy(o_t, lin_hbm.at[pl.ds(lane * 16, 2048)], wsem)
            pltpu.make_async_copy(o_t, lin_hbm.at[pl.ds(0, 2048)], wsem).wait()

        @pl.when(wid == 31)
        def _tail_partial():
            pltpu.sync_copy(tail_hbm, o_t.at[pl.ds(0, 1024)])
            pltpu.sync_copy(o_t.at[pl.ds(0, 1024)],
                            lin_hbm.at[pl.ds((_NTILE - 1) * 2048, 1024)])

    return relin


def _make_gather():
    mesh = plsc.VectorSubcoreMesh(core_axis_name="c", subcore_axis_name="s")
    n_out = _S * D_EMBEDDING * _B

    @functools.partial(
        pl.kernel,
        mesh=mesh,
        out_type=jax.ShapeDtypeStruct((n_out,), jnp.float32),
        compiler_params=pltpu.CompilerParams(
            use_tc_tiling_on_sc=False, needs_layout_passes=False),
        scratch_types=[
            pltpu.VMEM((_S, _LANE), jnp.int32),
            pltpu.VMEM((_GS * _LANE, D_EMBEDDING), jnp.float32),
            pltpu.VMEM((_GS * _LANE, D_EMBEDDING), jnp.float32),
            pltpu.VMEM((_GS * 2048,), jnp.float32),
            pltpu.VMEM((_GS * 2048,), jnp.float32),
            pltpu.SemaphoreType.DMA,
            pltpu.SemaphoreType.DMA,
        ],
    )
    def gather(table_hbm, idxt_hbm, out_hbm, idxv, r_a, r_b, stg_a, stg_b,
               gsem, wsem):
        wid = lax.axis_index("s") * _NC + lax.axis_index("c")
        iota = lax.iota(jnp.int32, 16)

        pltpu.sync_copy(idxt_hbm.at[:, pl.ds(wid * _LANE, _LANE)], idxv)

        def fire_g(g, rbuf):
            for j in range(_GS):
                pltpu.async_copy(
                    table_hbm.at[idxv.at[g * _GS + j]],
                    rbuf.at[pl.ds(j * _LANE, _LANE)], gsem)

        def drain_g(rbuf):
            for j in range(_GS):
                pltpu.make_async_copy(
                    table_hbm.at[idxv.at[0]],
                    rbuf.at[pl.ds(j * _LANE, _LANE)], gsem).wait()

        rows = [iota + c * 16 for c in range(8)]

        def transpose(rbuf, stg):
            @plsc.parallel_loop(0, _GS * D_EMBEDDING, 1, unroll=2)
            def _(i):
                j = i >> 4
                d = i & 15
                col = jnp.zeros((16,), jnp.int32) + d
                base = j * 2048 + d * 128
                joff = j << 7
                for c in range(8):
                    v = plsc.load_gather(rbuf, [rows[c] + joff, col])
                    stg[pl.ds(base + c * 16, 16)] = v

        def fire_w(g, stg):
            for j in range(_GS):
                s = g * _GS + j
                for dt in range(2):
                    pltpu.async_copy(
                        stg.at[pl.ds(j * 2048 + dt * 1024, 1024)],
                        out_hbm.at[pl.ds(((2 * s + dt) * 32 + wid) * 1024, 1024)],
                        wsem)

        def drain_w(stg):
            for j in range(_GS):
                for dt in range(2):
                    pltpu.make_async_copy(
                        stg.at[pl.ds(j * 2048 + dt * 1024, 1024)],
                        out_hbm.at[pl.ds(0, 1024)], wsem).wait()

        def process(g, rbuf, stg, fire_ahead, first):
            drain_g(rbuf)
            if not first:
                drain_w(stg)
            transpose(rbuf, stg)
            fire_w(g, stg)
            if fire_ahead:
                fire_g(g + 2, rbuf)

        fire_g(0, r_a)
        fire_g(1, r_b)
        process(0, r_a, stg_a, fire_ahead=True, first=True)
        process(1, r_b, stg_b, fire_ahead=True, first=True)

        def body(i, carry):
            process(2 * i, r_a, stg_a, fire_ahead=True, first=False)
            process(2 * i + 1, r_b, stg_b, fire_ahead=True, first=False)
            return carry

        lax.fori_loop(1, _NG // 2 - 1, body, 0)
        process(_NG - 2, r_a, stg_a, fire_ahead=False, first=False)
        process(_NG - 1, r_b, stg_b, fire_ahead=False, first=False)
        drain_w(stg_a)
        drain_w(stg_b)

    return gather


def kernel(x, table):
    idx_t = _quantize_t(x)
    tail = table[(_NTILE - 1) * 128:, :].reshape(1024)
    lin = _make_relinearize()(table.T, tail).reshape(_NROWS_PAD, 16)
    flat = _make_gather()(lin, idx_t)
    return (flat.reshape(_S, 2, 32, 8, _LANE)
            .transpose(2, 4, 0, 1, 3)
            .reshape(_B, _S, D_EMBEDDING))
